# Initial kernel scaffold; baseline (speedup 1.0000x reference)
import jax
import jax.numpy as jnp
from jax.experimental import pallas as pl


def kernel(nodes, adj_indices, adj_values, W1, b1, W2, W_skip, b2):
    raise NotImplementedError("write your pallas kernel here")


# speedup vs baseline: 3.5765x; 3.5765x over previous
import functools

import jax
import jax.numpy as jnp
from jax import lax
from jax.experimental import pallas as pl
from jax.experimental.pallas import tpu as pltpu
from jax.experimental.pallas import tpu_sc as plsc

N = 10000
D = 128
E = 320000
NC, NS, L = 2, 16, 16
NW = NC * NS
EB = 128
BATCHES = -(-E // (NW * EB))
PER_W = BATCHES * EB
E_PAD = PER_W * NW
N_ACC = 10240
ROWS_PER_TILE = N_ACC // NS

_mesh = plsc.VectorSubcoreMesh(core_axis_name="c", subcore_axis_name="s")


@functools.partial(
    pl.kernel,
    mesh=_mesh,
    out_type=jax.ShapeDtypeStruct((NC, N_ACC, D), jnp.float32),
    scratch_types=[
        pltpu.VMEM((EB,), jnp.int32),
        pltpu.VMEM((EB,), jnp.int32),
        pltpu.VMEM((EB,), jnp.float32),
        pltpu.VMEM((EB, D), jnp.float32),
        pltpu.VMEM_SHARED((N_ACC, D), jnp.float32),
        pltpu.SemaphoreType.DMA,
    ],
)
def _spmm_sc(x_hbm, rows_hbm, cols_hbm, vals_hbm, out_hbm,
             cols_v, rows_v, vals_v, gat_v, acc_sh, sem):
    cid = lax.axis_index("c")
    sid = lax.axis_index("s")
    wid = sid * NC + cid

    def _zrow(j, carry):
        for q in range(D // L):
            gat_v[j, pl.ds(q * L, L)] = jnp.zeros((L,), jnp.float32)
        return carry
    lax.fori_loop(0, EB, _zrow, 0)
    row0 = sid * ROWS_PER_TILE
    for k in range(ROWS_PER_TILE // EB):
        pltpu.sync_copy(gat_v, acc_sh.at[pl.ds(row0 + EB * k, EB)])
    plsc.subcore_barrier()

    def _batch(i, carry):
        base = wid * PER_W + i * EB
        pltpu.sync_copy(cols_hbm.at[pl.ds(base, EB)], cols_v)
        pltpu.sync_copy(rows_hbm.at[pl.ds(base, EB)], rows_v)
        pltpu.sync_copy(vals_hbm.at[pl.ds(base, EB)], vals_v)
        pltpu.async_copy(x_hbm.at[cols_v], gat_v, sem).wait()

        def _scale(g, c):
            vv = vals_v[pl.ds(g * L, L)]
            for t in range(L):
                v = jnp.full((L,), vv[t], jnp.float32)
                j = g * L + t
                for q in range(D // L):
                    sl = pl.ds(q * L, L)
                    gat_v[j, sl] = gat_v[j, sl] * v
            return c
        lax.fori_loop(0, EB // L, _scale, 0)
        pltpu.sync_copy(gat_v, acc_sh.at[rows_v], add=True)
        return carry
    lax.fori_loop(0, BATCHES, _batch, 0)

    plsc.subcore_barrier()
    pltpu.sync_copy(acc_sh.at[pl.ds(row0, ROWS_PER_TILE)],
                    out_hbm.at[cid, pl.ds(row0, ROWS_PER_TILE)])


_RB = 1000


def _stage_a_body(x_ref, w1_ref, ws_ref, b2_ref, xw1_ref, skip_ref):
    x = x_ref[...]
    xw1_ref[...] = jnp.dot(x, w1_ref[...], preferred_element_type=jnp.float32)
    skip_ref[...] = (jnp.dot(x, ws_ref[...], preferred_element_type=jnp.float32)
                     + b2_ref[...])


def _stage_b_body(p_ref, b1_ref, w2_ref, hw2_ref):
    h = jnp.maximum(p_ref[0] + p_ref[1] + b1_ref[...], 0.0)
    hw2_ref[...] = jnp.dot(h, w2_ref[...], preferred_element_type=jnp.float32)


def _stage_c_body(q_ref, skip_ref, out_ref):
    out_ref[...] = q_ref[0] + q_ref[1] + skip_ref[...]


def _stage_a(x, w1, ws, b2):
    return pl.pallas_call(
        _stage_a_body,
        grid=(N // _RB,),
        in_specs=[
            pl.BlockSpec((_RB, D), lambda i: (i, 0)),
            pl.BlockSpec((D, D), lambda i: (0, 0)),
            pl.BlockSpec((D, D), lambda i: (0, 0)),
            pl.BlockSpec((D,), lambda i: (0,)),
        ],
        out_specs=[
            pl.BlockSpec((_RB, D), lambda i: (i, 0)),
            pl.BlockSpec((_RB, D), lambda i: (i, 0)),
        ],
        out_shape=[
            jax.ShapeDtypeStruct((N, D), jnp.float32),
            jax.ShapeDtypeStruct((N, D), jnp.float32),
        ],
    )(x, w1, ws, b2)


def _stage_b(p, b1, w2):
    return pl.pallas_call(
        _stage_b_body,
        grid=(N // _RB,),
        in_specs=[
            pl.BlockSpec((NC, _RB, D), lambda i: (0, i, 0)),
            pl.BlockSpec((D,), lambda i: (0,)),
            pl.BlockSpec((D, D), lambda i: (0, 0)),
        ],
        out_specs=pl.BlockSpec((_RB, D), lambda i: (i, 0)),
        out_shape=jax.ShapeDtypeStruct((N, D), jnp.float32),
    )(p, b1, w2)


def _stage_c(q, skip):
    return pl.pallas_call(
        _stage_c_body,
        grid=(N // _RB,),
        in_specs=[
            pl.BlockSpec((NC, _RB, D), lambda i: (0, i, 0)),
            pl.BlockSpec((_RB, D), lambda i: (i, 0)),
        ],
        out_specs=pl.BlockSpec((_RB, D), lambda i: (i, 0)),
        out_shape=jax.ShapeDtypeStruct((N, D), jnp.float32),
    )(q, skip)


def kernel(nodes, adj_indices, adj_values, W1, b1, W2, W_skip, b2):
    pad = E_PAD - E
    rows = jnp.concatenate(
        [adj_indices[0].astype(jnp.int32), jnp.zeros((pad,), jnp.int32)])
    cols = jnp.concatenate(
        [adj_indices[1].astype(jnp.int32), jnp.zeros((pad,), jnp.int32)])
    vals = jnp.concatenate([adj_values, jnp.zeros((pad,), jnp.float32)])

    xw1, skip = _stage_a(nodes, W1, W_skip, b2)
    p = _spmm_sc(xw1, rows, cols, vals)
    hw2 = _stage_b(p, b1, W2)
    q = _spmm_sc(hw2, rows, cols, vals)
    return _stage_c(q, skip)
